# baseline (speedup 1.0000x reference)
import jax
import jax.numpy as jnp
from jax.experimental import pallas as pl
from jax.experimental.pallas import tpu as pltpu

_DELTA = 0.1
_NLAB = 8


def _edge_loss_kernel(pred_ref, edge_ref, elab_ref, out_ref):
    x = pred_ref[0]
    ev = edge_ref[0]
    edge_label = elab_ref[0, 0]

    m = jnp.max(x, axis=0, keepdims=True)
    e = jnp.exp(x - m)
    s = jnp.sum(e, axis=0, keepdims=True)
    p = e / s

    mu_pix = jnp.zeros_like(p)
    for v in range(_NLAB):
        onehot = ev == v
        cnt = jnp.sum(onehot.astype(jnp.float32))
        sum_v = jnp.sum(jnp.where(onehot[None], p, 0.0), axis=(1, 2),
                        keepdims=True)
        mu_v = sum_v / cnt
        mu_pix = jnp.where(onehot[None], mu_v, mu_pix)

    dist = jnp.sum(jnp.abs(p - mu_pix), axis=0)
    dist = jnp.maximum(dist - _DELTA, 0.0)

    mask = ((ev != edge_label) & (ev != 255)).astype(jnp.float32)
    l_var = jnp.sum(dist * mask) / (jnp.sum(mask) + 1e-5)
    out_ref[0, 0, 0] = l_var


def kernel(pred_sg_up, edge_v, edge_label):
    B, C, H, W = pred_sg_up.shape
    elab = jnp.asarray(edge_label, jnp.int32).reshape(1, 1)
    out = pl.pallas_call(
        _edge_loss_kernel,
        grid=(B,),
        in_specs=[
            pl.BlockSpec((1, C, H, W), lambda i: (i, 0, 0, 0)),
            pl.BlockSpec((1, H, W), lambda i: (i, 0, 0)),
            pl.BlockSpec((1, 1), lambda i: (0, 0), memory_space=pltpu.SMEM),
        ],
        out_specs=pl.BlockSpec((1, 1, 1), lambda i: (i, 0, 0),
                               memory_space=pltpu.SMEM),
        out_shape=jax.ShapeDtypeStruct((B, 1, 1), jnp.float32),
        compiler_params=pltpu.CompilerParams(
            dimension_semantics=("parallel",)),
    )(pred_sg_up, edge_v, elab)
    return jnp.sum(out) / B


# speedup vs baseline: 1.1565x; 1.1565x over previous
import jax
import jax.numpy as jnp
from jax.experimental import pallas as pl
from jax.experimental.pallas import tpu as pltpu

_DELTA = 0.1
_NLAB = 8


def _edge_loss_kernel(pred_ref, edge_ref, elab_ref, out_ref):
    x = pred_ref[0]
    ev0 = edge_ref[0]
    edge_label = elab_ref[0, 0]

    ev = ev0 ^ edge_label

    m = jnp.max(x, axis=0, keepdims=True)
    e = jnp.exp(x - m)
    s = jnp.sum(e, axis=0, keepdims=True)
    p = e / s

    mu_pix = jnp.zeros_like(p)
    for v in range(1, _NLAB):
        onehot = ev == v
        ohf = onehot.astype(jnp.float32)
        cnt = jnp.sum(ohf)
        sum_v = jnp.sum(p * ohf[None], axis=(1, 2),
                        keepdims=True)
        mu_v = sum_v / cnt
        mu_pix = jnp.where(onehot[None], mu_v, mu_pix)

    dist = jnp.sum(jnp.abs(p - mu_pix), axis=0)
    dist = jnp.maximum(dist - _DELTA, 0.0)

    mask = ((ev != 0) & (ev0 != 255)).astype(jnp.float32)
    l_var = jnp.sum(dist * mask) / (jnp.sum(mask) + 1e-5)
    out_ref[0, 0, 0] = l_var


def kernel(pred_sg_up, edge_v, edge_label):
    B, C, H, W = pred_sg_up.shape
    elab = jnp.asarray(edge_label, jnp.int32).reshape(1, 1)
    out = pl.pallas_call(
        _edge_loss_kernel,
        grid=(B,),
        in_specs=[
            pl.BlockSpec((1, C, H, W), lambda i: (i, 0, 0, 0)),
            pl.BlockSpec((1, H, W), lambda i: (i, 0, 0)),
            pl.BlockSpec((1, 1), lambda i: (0, 0), memory_space=pltpu.SMEM),
        ],
        out_specs=pl.BlockSpec((1, 1, 1), lambda i: (i, 0, 0),
                               memory_space=pltpu.SMEM),
        out_shape=jax.ShapeDtypeStruct((B, 1, 1), jnp.float32),
        compiler_params=pltpu.CompilerParams(
            dimension_semantics=("parallel",)),
    )(pred_sg_up, edge_v, elab)
    return jnp.sum(out) / B
